# Initial kernel scaffold; baseline (speedup 1.0000x reference)
import jax
import jax.numpy as jnp
from jax.experimental import pallas as pl


def kernel(p0_continuous, p0_binary, p0_controller, p0_action, p0_jumps, p1_continuous, p1_binary, p1_controller, p1_action, p1_jumps, action_table, jumps_table):
    raise NotImplementedError("write your pallas kernel here")


# speedup vs baseline: 2.0162x; 2.0162x over previous
import functools

import jax
import jax.numpy as jnp
from jax import lax
from jax.experimental import pallas as pl
from jax.experimental.pallas import tpu as pltpu
from jax.experimental.pallas import tpu_sc as plsc

B = 16384
OUT_D = 112
NC = 2
NS = 16
NW = NC * NS
RPW = B // NW
L = 16

_mesh = plsc.VectorSubcoreMesh(core_axis_name="c", subcore_axis_name="s")


@functools.partial(
    pl.kernel,
    out_type=jax.ShapeDtypeStruct((B, OUT_D), jnp.float32),
    mesh=_mesh,
    scratch_types=[
        pltpu.VMEM((RPW,), jnp.int32),
        pltpu.VMEM((RPW,), jnp.int32),
        pltpu.VMEM((RPW,), jnp.int32),
        pltpu.VMEM((RPW,), jnp.int32),
        pltpu.VMEM((RPW, 4), jnp.float32),
        pltpu.VMEM((RPW, 4), jnp.float32),
        pltpu.VMEM((RPW, 3), jnp.float32),
        pltpu.VMEM((RPW, 3), jnp.float32),
        pltpu.VMEM((RPW, 13), jnp.float32),
        pltpu.VMEM((RPW, 13), jnp.float32),
        pltpu.VMEM((8, 4), jnp.float32),
        pltpu.VMEM((RPW, 40), jnp.float32),
        pltpu.VMEM((RPW, 40), jnp.float32),
        pltpu.VMEM((RPW, 16), jnp.float32),
        pltpu.VMEM((RPW, 16), jnp.float32),
        pltpu.SemaphoreType.DMA,
    ],
    compiler_params=pltpu.CompilerParams(use_tc_tiling_on_sc=False,
                                         needs_layout_passes=False),
)
def _encode(p0c, p0b, p0k, p0a, p0j,
            p1c, p1b, p1k, p1a, p1j,
            at_p, jt, out,
            i0a, i0j, i1a, i1j,
            cs0, cs1, b0s, b1s, k0s, k1s, jt_s,
            ea0, ea1, f0, f1, sem):
    wid = lax.axis_index("s") * NC + lax.axis_index("c")
    base = wid * RPW
    sl = pl.ds(base, RPW)

    pltpu.sync_copy(p0a.at[sl], i0a)
    pltpu.sync_copy(p1a.at[sl], i1a)
    pltpu.sync_copy(p0j.at[sl], i0j)
    pltpu.sync_copy(p1j.at[sl], i1j)
    pltpu.sync_copy(p0c.at[sl], cs0)
    pltpu.sync_copy(p1c.at[sl], cs1)
    pltpu.sync_copy(p0b.at[sl], b0s)
    pltpu.sync_copy(p1b.at[sl], b1s)
    pltpu.sync_copy(p0k.at[sl], k0s)
    pltpu.sync_copy(p1k.at[sl], k1s)
    pltpu.sync_copy(jt, jt_s)

    a0 = pltpu.async_copy(at_p.at[i0a], ea0, sem)
    a1 = pltpu.async_copy(at_p.at[i1a], ea1, sem)
    a0.wait()
    a1.wait()

    lanes = lax.iota(jnp.int32, L)

    def body(g, _):
        rv = lanes + g * L
        jidx0 = i0j[pl.ds(g * L, L)]
        jidx1 = i1j[pl.ds(g * L, L)]
        for src, dst, js, jd, w in (
            (cs0, f0, 0, 0, 4), (b0s, f0, 0, 4, 3), (k0s, f0, 0, 7, 9),
            (cs1, f1, 0, 0, 4), (b1s, f1, 0, 4, 3), (k1s, f1, 0, 7, 9),
            (k0s, ea0, 9, 0, 4), (k1s, ea1, 9, 0, 4),
        ):
            for j in range(w):
                jv = jnp.full((L,), j, jnp.int32)
                plsc.store_scatter(dst, [rv, jv + jd],
                                   plsc.load_gather(src, [rv, jv + js]))
        for j in range(4):
            jv = jnp.full((L,), j, jnp.int32)
            plsc.store_scatter(ea0, [rv, jv + 36],
                               plsc.load_gather(jt_s, [jidx0, jv]))
            plsc.store_scatter(ea1, [rv, jv + 36],
                               plsc.load_gather(jt_s, [jidx1, jv]))
        return ()

    lax.fori_loop(0, RPW // L, body, ())

    pltpu.sync_copy(f0, out.at[sl, pl.ds(0, 16)])
    pltpu.sync_copy(ea0, out.at[sl, pl.ds(16, 40)])
    pltpu.sync_copy(f1, out.at[sl, pl.ds(56, 16)])
    pltpu.sync_copy(ea1, out.at[sl, pl.ds(72, 40)])


def kernel(p0_continuous, p0_binary, p0_controller, p0_action, p0_jumps,
           p1_continuous, p1_binary, p1_controller, p1_action, p1_jumps,
           action_table, jumps_table):
    at_p = jnp.pad(action_table, ((0, 0), (4, 4)))
    return _encode(p0_continuous, p0_binary, p0_controller,
                   p0_action.astype(jnp.int32), p0_jumps.astype(jnp.int32),
                   p1_continuous, p1_binary, p1_controller,
                   p1_action.astype(jnp.int32), p1_jumps.astype(jnp.int32),
                   at_p, jumps_table)
